# baseline (speedup 1.0000x reference)
import jax
import jax.numpy as jnp
from jax.experimental import pallas as pl

B = 2
N = 8192
K = 32
NUM_CLASSES = 3


def _in_conv_kernel(c_ref, w_ref, o_ref):
    o_ref[...] = c_ref[...] @ w_ref[...]


def _ball_query(coords, feats, center_idx, max_radius, k):
    n_src = feats.shape[1]
    src = coords[:, :n_src, :]
    centers = jax.vmap(lambda c, i: c[i])(src, center_idx)
    d2 = jnp.sum((centers[:, :, None, :] - src[:, None, :, :]) ** 2, axis=-1)
    neg, idx = jax.lax.top_k(-d2, k)
    d_sel = -neg
    valid = d_sel <= max_radius ** 2
    idx = jnp.where(valid, idx, idx[:, :, :1])
    grouped = jax.vmap(lambda f, i: f[i])(feats, idx)
    return grouped


def _adaptive_agg(grouped, Wt, Wa):
    h = jax.nn.relu(grouped @ Wt)
    attn = jax.nn.softmax((h @ Wa)[..., 0], axis=-1)
    return jnp.sum(h * attn[..., None], axis=2)


def _csit(x_low, x_high, W1, W2):
    g = jnp.mean(x_high, axis=1)
    g = jax.nn.sigmoid(jax.nn.relu(g @ W1) @ W2)
    return x_low * g[:, None, :] + x_high


def _upsample_nearest(x, n_out):
    idx = (jnp.arange(n_out) * x.shape[1]) // n_out
    return x[:, idx, :]


def _batchnorm(x, gamma, beta, eps=1e-5):
    mu = jnp.mean(x, axis=(0, 1), keepdims=True)
    var = jnp.var(x, axis=(0, 1), keepdims=True)
    return (x - mu) / jnp.sqrt(var + eps) * gamma + beta


def kernel(points, center_idx1, center_idx2, W_in, Wt1, Wa1, W_down1, W_red1,
           Wt2, Wa2, W_down2, W_red2, C1_W1, C1_W2, C2_W1, C2_W2,
           W_out1, bn_gamma, bn_beta, W_out2, b_out2):
    coords = points[:, :, :3]
    x0 = pl.pallas_call(
        _in_conv_kernel,
        out_shape=jax.ShapeDtypeStruct((B, N, 64), jnp.float32),
        grid=(B,),
        in_specs=[
            pl.BlockSpec((1, N, 3), lambda b: (b, 0, 0)),
            pl.BlockSpec((3, 64), lambda b: (0, 0)),
        ],
        out_specs=pl.BlockSpec((1, N, 64), lambda b: (b, 0, 0)),
    )(coords, W_in)
    g1 = _ball_query(coords, x0, center_idx1, 0.3, K)
    x1 = _adaptive_agg(g1, Wt1, Wa1)
    x1 = (x1 @ W_down1) @ W_red1
    g2 = _ball_query(coords, x1, center_idx2, 0.5, K)
    x2 = _adaptive_agg(g2, Wt2, Wa2)
    x2 = (x2 @ W_down2) @ W_red2
    x1_up = _upsample_nearest(x1, N)
    x1_enh = _csit(x0, x1_up, C1_W1, C1_W2)
    x2_up = _upsample_nearest(x2, N)
    x0_enh = _csit(x1_enh, x2_up, C2_W1, C2_W2)
    fused = jnp.concatenate([x0_enh, x1_up, x2_up], axis=-1)
    h = jax.nn.relu(_batchnorm(fused @ W_out1, bn_gamma, bn_beta))
    logits = h @ W_out2 + b_out2
    return logits


# speedup vs baseline: 14.6387x; 14.6387x over previous
import functools

import jax
import jax.numpy as jnp
from jax.experimental import pallas as pl
from jax.experimental.pallas import tpu as pltpu
from jax.experimental.pallas import tpu_sc as plsc

B = 2
N = 8192
KNN = 32
TABW = 128

def _dot(a, b):
    return jnp.dot(a, b, preferred_element_type=jnp.float32,
                   precision=jax.lax.Precision.HIGHEST)


def _bdot(a, b):
    return jnp.dot(a.astype(jnp.bfloat16), b.astype(jnp.bfloat16),
                   preferred_element_type=jnp.float32)


def _prologue_body(coords_ref, win_ref, wt1_ref, wa1_ref,
                   x0_ref, h0_ref, s0_ref):
    coords = coords_ref[0]
    x0 = _bdot(coords, win_ref[...])
    h0 = jax.nn.relu(_bdot(x0, wt1_ref[...]))
    s0 = _bdot(h0, wa1_ref[...])
    x0_ref[0] = x0
    h0_ref[0] = h0
    s0_ref[0] = s0


def _prologue(coords, W_in, Wt1, Wa1):
    return pl.pallas_call(
        _prologue_body,
        out_shape=(
            jax.ShapeDtypeStruct((B, N, 64), jnp.float32),
            jax.ShapeDtypeStruct((B, N, 128), jnp.float32),
            jax.ShapeDtypeStruct((B, N, 1), jnp.float32),
        ),
        grid=(B,),
        in_specs=[
            pl.BlockSpec((1, N, 3), lambda b: (b, 0, 0)),
            pl.BlockSpec((3, 64), lambda b: (0, 0)),
            pl.BlockSpec((64, 128), lambda b: (0, 0)),
            pl.BlockSpec((128, 1), lambda b: (0, 0)),
        ],
        out_specs=(
            pl.BlockSpec((1, N, 64), lambda b: (b, 0, 0)),
            pl.BlockSpec((1, N, 128), lambda b: (b, 0, 0)),
            pl.BlockSpec((1, N, 1), lambda b: (b, 0, 0)),
        ),
    )(coords, W_in, Wt1, Wa1)


def _sc_gather_rows(table, idx):
    num = idx.shape[0]
    win = 128
    idx2 = idx.reshape(1, num)
    mesh = plsc.VectorSubcoreMesh(core_axis_name="c", subcore_axis_name="s")

    @functools.partial(
        pl.kernel,
        out_type=jax.ShapeDtypeStruct((num, TABW), jnp.float32),
        mesh=mesh,
    )
    def k(tab_hbm, i_hbm, o_hbm):
        def body(i_vmem, o_vmem):
            pltpu.sync_copy(tab_hbm.at[i_vmem.at[0]], o_vmem)

        pltpu.emit_pipeline(
            body,
            grid=(num // win,),
            in_specs=[pl.BlockSpec((1, win), index_map=lambda i: (0, i))],
            out_specs=[pl.BlockSpec((win, TABW), index_map=lambda i: (i, 0))],
            core_axis_name=("c", "s"),
            dimension_semantics=(pltpu.PARALLEL,),
        )(i_hbm, o_hbm)

    return k(table, idx2)


def _gather_centers(coords_part, s_col, center_idx):
    ns = coords_part.shape[1]
    m = center_idx.shape[1]
    pad = jnp.zeros((B, ns, TABW - 4), jnp.float32)
    table = jnp.concatenate([coords_part, s_col, pad], axis=-1)
    table = table.reshape(B * ns, TABW)
    idx = (center_idx.astype(jnp.int32)
           + (jnp.arange(B, dtype=jnp.int32) * ns)[:, None]).reshape(-1)
    out = _sc_gather_rows(table, idx)
    return out.reshape(B, m, TABW)


def _knn_weights(bits, cent, s_row, r2):
    mb, ns = bits.shape

    def srch(i, acc):
        cand = acc | jax.lax.shift_left(jnp.int32(1), jnp.int32(30) - i)
        cnt = jnp.sum((bits < cand).astype(jnp.int32), axis=1, keepdims=True)
        return jnp.where(cnt < KNN, cand, acc)

    tstar = jax.lax.fori_loop(0, 31, srch, jnp.zeros((mb, 1), jnp.int32))

    r2bits = jnp.float32(r2).view(jnp.int32)
    sel = bits <= tstar
    val = sel & (bits <= r2bits)

    neg = jnp.float32(-3.4e38)
    mx = jnp.max(jnp.where(val, s_row, neg), axis=1, keepdims=True)
    ew = jnp.where(val, jnp.exp(s_row - mx), 0.0)
    sume = jnp.sum(ew, axis=1, keepdims=True)
    cnt_val = jnp.sum(val.astype(jnp.float32), axis=1, keepdims=True)
    nv = jnp.maximum(jnp.float32(KNN) - cnt_val, 0.0)

    s0c = cent[:, 3:4]
    ec = jnp.exp(s0c - mx) * nv
    denom = sume + ec
    invd = 1.0 / denom

    iota = jax.lax.broadcasted_iota(jnp.int32, (mb, ns), 1)
    cidx = jnp.min(jnp.where(bits == 0, iota, jnp.int32(ns)), axis=1,
                   keepdims=True)
    w = ew * invd + jnp.where(iota == cidx, ec * invd, 0.0)
    return w


def _d2bits(cent, src_t):
    cx = cent[:, 0:1]
    cy = cent[:, 1:2]
    cz = cent[:, 2:3]
    dx = src_t[0:1, :] - cx
    dy = src_t[1:2, :] - cy
    dz = src_t[2:3, :] - cz
    d2 = dx * dx + dy * dy + dz * dz
    return d2.view(jnp.int32)


def _stage1_body(srcT_ref, cent_ref, s0row_ref, h0_ref,
                 wd_ref, wr_ref, wt2_ref, wa2_ref,
                 x1_ref, h1_ref, s1_ref):
    cent = cent_ref[0]
    bits = _d2bits(cent, srcT_ref[0])
    s_row = s0row_ref[0]
    w = _knn_weights(bits, cent, s_row, 0.3 ** 2)
    agg = _dot(w, h0_ref[0])
    x1 = _bdot(_bdot(agg, wd_ref[...]), wr_ref[...])
    h1 = jax.nn.relu(_bdot(x1, wt2_ref[...]))
    s1 = _bdot(h1, wa2_ref[...])
    x1_ref[0] = x1
    h1_ref[0] = h1
    s1_ref[0] = s1


def _stage2_body(srcT_ref, cent_ref, s1row_ref, h1_ref,
                 wd_ref, wr_ref, x2_ref):
    cent = cent_ref[0]
    bits = _d2bits(cent, srcT_ref[0])
    s_row = s1row_ref[0]
    w = _knn_weights(bits, cent, s_row, 0.5 ** 2)
    agg = _dot(w, h1_ref[0])
    x2_ref[0] = _bdot(_bdot(agg, wd_ref[...]), wr_ref[...])


def _stage1(srcT, cent1, s0row, H0, W_down1, W_red1, Wt2, Wa2, mb=256):
    m = N // 2
    return pl.pallas_call(
        _stage1_body,
        out_shape=(
            jax.ShapeDtypeStruct((B, m, 64), jnp.float32),
            jax.ShapeDtypeStruct((B, m, 256), jnp.float32),
            jax.ShapeDtypeStruct((B, m, 1), jnp.float32),
        ),
        grid=(B, m // mb),
        in_specs=[
            pl.BlockSpec((1, 8, N), lambda b, i: (b, 0, 0)),
            pl.BlockSpec((1, mb, TABW), lambda b, i: (b, i, 0)),
            pl.BlockSpec((1, 1, N), lambda b, i: (b, 0, 0)),
            pl.BlockSpec((1, N, 128), lambda b, i: (b, 0, 0)),
            pl.BlockSpec((128, 128), lambda b, i: (0, 0)),
            pl.BlockSpec((128, 64), lambda b, i: (0, 0)),
            pl.BlockSpec((64, 256), lambda b, i: (0, 0)),
            pl.BlockSpec((256, 1), lambda b, i: (0, 0)),
        ],
        out_specs=(
            pl.BlockSpec((1, mb, 64), lambda b, i: (b, i, 0)),
            pl.BlockSpec((1, mb, 256), lambda b, i: (b, i, 0)),
            pl.BlockSpec((1, mb, 1), lambda b, i: (b, i, 0)),
        ),
    )(srcT, cent1, s0row, H0, W_down1, W_red1, Wt2, Wa2)


def _stage2(srcT2, cent2, s1row, H1, W_down2, W_red2, mb=256):
    m = N // 4
    ns = N // 2
    return pl.pallas_call(
        _stage2_body,
        out_shape=jax.ShapeDtypeStruct((B, m, 64), jnp.float32),
        grid=(B, m // mb),
        in_specs=[
            pl.BlockSpec((1, 8, ns), lambda b, i: (b, 0, 0)),
            pl.BlockSpec((1, mb, TABW), lambda b, i: (b, i, 0)),
            pl.BlockSpec((1, 1, ns), lambda b, i: (b, 0, 0)),
            pl.BlockSpec((1, ns, 256), lambda b, i: (b, 0, 0)),
            pl.BlockSpec((256, 256), lambda b, i: (0, 0)),
            pl.BlockSpec((256, 64), lambda b, i: (0, 0)),
        ],
        out_specs=pl.BlockSpec((1, mb, 64), lambda b, i: (b, i, 0)),
    )(srcT2, cent2, s1row, H1, W_down2, W_red2)


def _gate_body(x1_ref, x2_ref, c1w1_ref, c1w2_ref, c2w1_ref, c2w2_ref,
               s1_ref, s2_ref):
    x1u = jnp.repeat(x1_ref[...], 2, axis=1)
    x2u = jnp.repeat(x2_ref[...], 4, axis=1)
    g1 = jnp.mean(x1u, axis=1)
    s1_ref[...] = jax.nn.sigmoid(
        _bdot(jax.nn.relu(_bdot(g1, c1w1_ref[...])), c1w2_ref[...]))
    g2 = jnp.mean(x2u, axis=1)
    s2_ref[...] = jax.nn.sigmoid(
        _bdot(jax.nn.relu(_bdot(g2, c2w1_ref[...])), c2w2_ref[...]))


def _gates(x1, x2, C1_W1, C1_W2, C2_W1, C2_W2):
    full = lambda *shape: pl.BlockSpec(shape, lambda: tuple(0 for _ in shape))
    return pl.pallas_call(
        _gate_body,
        out_shape=(jax.ShapeDtypeStruct((B, 64), jnp.float32),
                   jax.ShapeDtypeStruct((B, 64), jnp.float32)),
        grid=(),
        in_specs=[full(B, N // 2, 64), full(B, N // 4, 64),
                  full(64, 64), full(64, 64), full(64, 64), full(64, 64)],
        out_specs=(full(B, 64), full(B, 64)),
    )(x1, x2, C1_W1, C1_W2, C2_W1, C2_W2)


def _epi_y_body(x0_ref, x1_ref, x2_ref, s1_ref, s2_ref, wo1_ref,
                y_ref, ysum_ref, ysq_ref):
    b = pl.program_id(0)
    i = pl.program_id(1)
    x0 = x0_ref[0]
    x1u = jnp.repeat(x1_ref[0], 2, axis=0)
    x2u = jnp.repeat(x2_ref[0], 4, axis=0)
    sig1 = s1_ref[pl.ds(b, 1), :]
    sig2 = s2_ref[pl.ds(b, 1), :]
    x1e = x0 * sig1 + x1u
    x0e = x1e * sig2 + x2u
    fused = jnp.concatenate([x0e, x1u, x2u], axis=1)
    y = _bdot(fused, wo1_ref[...])
    y_ref[0] = y

    @pl.when(jnp.logical_and(b == 0, i == 0))
    def _():
        ysum_ref[...] = jnp.zeros_like(ysum_ref)
        ysq_ref[...] = jnp.zeros_like(ysq_ref)

    ysum_ref[...] += jnp.sum(y, axis=0, keepdims=True)
    ysq_ref[...] += jnp.sum(y * y, axis=0, keepdims=True)


def _epi_y(x0, x1, x2, sig1, sig2, W_out1, nb=2048):
    full = lambda *shape: pl.BlockSpec(shape, lambda b, i: tuple(0 for _ in shape))
    return pl.pallas_call(
        _epi_y_body,
        out_shape=(jax.ShapeDtypeStruct((B, N, 256), jnp.float32),
                   jax.ShapeDtypeStruct((1, 256), jnp.float32),
                   jax.ShapeDtypeStruct((1, 256), jnp.float32)),
        grid=(B, N // nb),
        in_specs=[
            pl.BlockSpec((1, nb, 64), lambda b, i: (b, i, 0)),
            pl.BlockSpec((1, nb // 2, 64), lambda b, i: (b, i, 0)),
            pl.BlockSpec((1, nb // 4, 64), lambda b, i: (b, i, 0)),
            full(B, 64), full(B, 64), full(192, 256),
        ],
        out_specs=(
            pl.BlockSpec((1, nb, 256), lambda b, i: (b, i, 0)),
            pl.BlockSpec((1, 256), lambda b, i: (0, 0)),
            pl.BlockSpec((1, 256), lambda b, i: (0, 0)),
        ),
    )(x0, x1, x2, sig1, sig2, W_out1)


def _epi_out_body(y_ref, ysum_ref, ysq_ref, gam_ref, bet_ref,
                  wo2_ref, bo2_ref, out_ref):
    cnt = jnp.float32(B * N)
    mu = ysum_ref[...] / cnt
    var = ysq_ref[...] / cnt - mu * mu
    y = y_ref[0]
    h = (y - mu) / jnp.sqrt(var + 1e-5) * gam_ref[...] + bet_ref[...]
    h = jax.nn.relu(h)
    out_ref[0] = _bdot(h, wo2_ref[...]) + bo2_ref[...]


def _epi_out(y, ysum, ysq, bn_gamma, bn_beta, W_out2, b_out2, nb=2048):
    nc = W_out2.shape[1]
    full = lambda *shape: pl.BlockSpec(shape, lambda b, i: tuple(0 for _ in shape))
    return pl.pallas_call(
        _epi_out_body,
        out_shape=jax.ShapeDtypeStruct((B, N, nc), jnp.float32),
        grid=(B, N // nb),
        in_specs=[
            pl.BlockSpec((1, nb, 256), lambda b, i: (b, i, 0)),
            full(1, 256), full(1, 256), full(1, 256), full(1, 256),
            full(256, nc), full(1, nc),
        ],
        out_specs=pl.BlockSpec((1, nb, nc), lambda b, i: (b, i, 0)),
    )(y, ysum, ysq, bn_gamma.reshape(1, 256), bn_beta.reshape(1, 256),
      W_out2, b_out2.reshape(1, nc))


def _epilogue(x0, x1, x2, C1_W1, C1_W2, C2_W1, C2_W2,
              W_out1, bn_gamma, bn_beta, W_out2, b_out2):
    sig1, sig2 = _gates(x1, x2, C1_W1, C1_W2, C2_W1, C2_W2)
    y, ysum, ysq = _epi_y(x0, x1, x2, sig1, sig2, W_out1)
    return _epi_out(y, ysum, ysq, bn_gamma, bn_beta, W_out2, b_out2)


def kernel(points, center_idx1, center_idx2, W_in, Wt1, Wa1, W_down1, W_red1,
           Wt2, Wa2, W_down2, W_red2, C1_W1, C1_W2, C2_W1, C2_W2,
           W_out1, bn_gamma, bn_beta, W_out2, b_out2):
    coords = points[:, :, :3]
    x0, H0, S0col = _prologue(coords, W_in, Wt1, Wa1)

    srcT = jnp.pad(jnp.swapaxes(coords, 1, 2), ((0, 0), (0, 5), (0, 0)))
    s0row = jnp.swapaxes(S0col, 1, 2)

    cent1 = _gather_centers(coords, S0col, center_idx1)
    x1, H1, S1col = _stage1(srcT, cent1, s0row, H0,
                            W_down1, W_red1, Wt2, Wa2)

    srcT2 = srcT[:, :, :N // 2]
    s1row = jnp.swapaxes(S1col, 1, 2)
    cent2 = _gather_centers(coords[:, :N // 2], S1col, center_idx2)
    x2 = _stage2(srcT2, cent2, s1row, H1, W_down2, W_red2)

    return _epilogue(x0, x1, x2, C1_W1, C1_W2, C2_W1, C2_W2,
                     W_out1, bn_gamma, bn_beta, W_out2, b_out2)


# speedup vs baseline: 15.5516x; 1.0624x over previous
import functools

import jax
import jax.numpy as jnp
from jax.experimental import pallas as pl
from jax.experimental.pallas import tpu as pltpu
from jax.experimental.pallas import tpu_sc as plsc

B = 2
N = 8192
KNN = 32
TABW = 128

def _dot(a, b):
    ah = a.astype(jnp.bfloat16)
    al = (a - ah.astype(jnp.float32)).astype(jnp.bfloat16)
    bh = b.astype(jnp.bfloat16)
    bl = (b - bh.astype(jnp.float32)).astype(jnp.bfloat16)
    mm = lambda x, y: jnp.dot(x, y, preferred_element_type=jnp.float32)
    return mm(ah, bh) + (mm(ah, bl) + mm(al, bh))


def _bdot(a, b):
    return jnp.dot(a.astype(jnp.bfloat16), b.astype(jnp.bfloat16),
                   preferred_element_type=jnp.float32)


def _prologue_body(coords_ref, win_ref, wt1_ref, wa1_ref,
                   x0_ref, h0_ref, s0_ref):
    coords = coords_ref[0]
    x0 = _bdot(coords, win_ref[...])
    h0 = jax.nn.relu(_bdot(x0, wt1_ref[...]))
    s0 = _bdot(h0, wa1_ref[...])
    x0_ref[0] = x0
    h0_ref[0] = h0
    s0_ref[0] = s0


def _prologue(coords, W_in, Wt1, Wa1):
    return pl.pallas_call(
        _prologue_body,
        out_shape=(
            jax.ShapeDtypeStruct((B, N, 64), jnp.float32),
            jax.ShapeDtypeStruct((B, N, 128), jnp.float32),
            jax.ShapeDtypeStruct((B, N, 1), jnp.float32),
        ),
        grid=(B,),
        in_specs=[
            pl.BlockSpec((1, N, 3), lambda b: (b, 0, 0)),
            pl.BlockSpec((3, 64), lambda b: (0, 0)),
            pl.BlockSpec((64, 128), lambda b: (0, 0)),
            pl.BlockSpec((128, 1), lambda b: (0, 0)),
        ],
        out_specs=(
            pl.BlockSpec((1, N, 64), lambda b: (b, 0, 0)),
            pl.BlockSpec((1, N, 128), lambda b: (b, 0, 0)),
            pl.BlockSpec((1, N, 1), lambda b: (b, 0, 0)),
        ),
    )(coords, W_in, Wt1, Wa1)


def _sc_gather_rows(table, idx):
    num = idx.shape[0]
    win = 128
    idx2 = idx.reshape(1, num)
    mesh = plsc.VectorSubcoreMesh(core_axis_name="c", subcore_axis_name="s")

    @functools.partial(
        pl.kernel,
        out_type=jax.ShapeDtypeStruct((num, TABW), jnp.float32),
        mesh=mesh,
    )
    def k(tab_hbm, i_hbm, o_hbm):
        def body(i_vmem, o_vmem):
            pltpu.sync_copy(tab_hbm.at[i_vmem.at[0]], o_vmem)

        pltpu.emit_pipeline(
            body,
            grid=(num // win,),
            in_specs=[pl.BlockSpec((1, win), index_map=lambda i: (0, i))],
            out_specs=[pl.BlockSpec((win, TABW), index_map=lambda i: (i, 0))],
            core_axis_name=("c", "s"),
            dimension_semantics=(pltpu.PARALLEL,),
        )(i_hbm, o_hbm)

    return k(table, idx2)


def _gather_centers(coords_part, s_col, center_idx):
    ns = coords_part.shape[1]
    m = center_idx.shape[1]
    pad = jnp.zeros((B, ns, TABW - 4), jnp.float32)
    table = jnp.concatenate([coords_part, s_col, pad], axis=-1)
    table = table.reshape(B * ns, TABW)
    idx = (center_idx.astype(jnp.int32)
           + (jnp.arange(B, dtype=jnp.int32) * ns)[:, None]).reshape(-1)
    out = _sc_gather_rows(table, idx)
    return out.reshape(B, m, TABW)


def _knn_weights(bits, cent, s_row, r2):
    mb, ns = bits.shape

    def srch(i, acc):
        cand = acc | jax.lax.shift_left(jnp.int32(1), jnp.int32(30) - i)
        cnt = jnp.sum((bits < cand).astype(jnp.int32), axis=1, keepdims=True)
        return jnp.where(cnt < KNN, cand, acc)

    tstar = jax.lax.fori_loop(0, 31, srch, jnp.zeros((mb, 1), jnp.int32))

    r2bits = jnp.float32(r2).view(jnp.int32)
    sel = bits <= tstar
    val = sel & (bits <= r2bits)

    neg = jnp.float32(-3.4e38)
    mx = jnp.max(jnp.where(val, s_row, neg), axis=1, keepdims=True)
    ew = jnp.where(val, jnp.exp(s_row - mx), 0.0)
    sume = jnp.sum(ew, axis=1, keepdims=True)
    cnt_val = jnp.sum(val.astype(jnp.float32), axis=1, keepdims=True)
    nv = jnp.maximum(jnp.float32(KNN) - cnt_val, 0.0)

    s0c = cent[:, 3:4]
    ec = jnp.exp(s0c - mx) * nv
    denom = sume + ec
    invd = 1.0 / denom

    iota = jax.lax.broadcasted_iota(jnp.int32, (mb, ns), 1)
    cidx = jnp.min(jnp.where(bits == 0, iota, jnp.int32(ns)), axis=1,
                   keepdims=True)
    w = ew * invd + jnp.where(iota == cidx, ec * invd, 0.0)
    return w


def _d2bits(cent, src_t):
    cx = cent[:, 0:1]
    cy = cent[:, 1:2]
    cz = cent[:, 2:3]
    dx = src_t[0:1, :] - cx
    dy = src_t[1:2, :] - cy
    dz = src_t[2:3, :] - cz
    d2 = dx * dx + dy * dy + dz * dz
    return d2.view(jnp.int32)


def _stage1_body(srcT_ref, cent_ref, s0row_ref, h0_ref,
                 wd_ref, wr_ref, wt2_ref, wa2_ref,
                 x1_ref, h1_ref, s1_ref):
    cent = cent_ref[0]
    bits = _d2bits(cent, srcT_ref[0])
    s_row = s0row_ref[0]
    w = _knn_weights(bits, cent, s_row, 0.3 ** 2)
    agg = _dot(w, h0_ref[0])
    x1 = _bdot(_bdot(agg, wd_ref[...]), wr_ref[...])
    h1 = jax.nn.relu(_bdot(x1, wt2_ref[...]))
    s1 = _bdot(h1, wa2_ref[...])
    x1_ref[0] = x1
    h1_ref[0] = h1
    s1_ref[0] = s1


def _stage2_body(srcT_ref, cent_ref, s1row_ref, h1_ref,
                 wd_ref, wr_ref, x2_ref):
    cent = cent_ref[0]
    bits = _d2bits(cent, srcT_ref[0])
    s_row = s1row_ref[0]
    w = _knn_weights(bits, cent, s_row, 0.5 ** 2)
    agg = _dot(w, h1_ref[0])
    x2_ref[0] = _bdot(_bdot(agg, wd_ref[...]), wr_ref[...])


def _stage1(srcT, cent1, s0row, H0, W_down1, W_red1, Wt2, Wa2, mb=256):
    m = N // 2
    return pl.pallas_call(
        _stage1_body,
        out_shape=(
            jax.ShapeDtypeStruct((B, m, 64), jnp.float32),
            jax.ShapeDtypeStruct((B, m, 256), jnp.float32),
            jax.ShapeDtypeStruct((B, m, 1), jnp.float32),
        ),
        grid=(B, m // mb),
        in_specs=[
            pl.BlockSpec((1, 8, N), lambda b, i: (b, 0, 0)),
            pl.BlockSpec((1, mb, TABW), lambda b, i: (b, i, 0)),
            pl.BlockSpec((1, 1, N), lambda b, i: (b, 0, 0)),
            pl.BlockSpec((1, N, 128), lambda b, i: (b, 0, 0)),
            pl.BlockSpec((128, 128), lambda b, i: (0, 0)),
            pl.BlockSpec((128, 64), lambda b, i: (0, 0)),
            pl.BlockSpec((64, 256), lambda b, i: (0, 0)),
            pl.BlockSpec((256, 1), lambda b, i: (0, 0)),
        ],
        out_specs=(
            pl.BlockSpec((1, mb, 64), lambda b, i: (b, i, 0)),
            pl.BlockSpec((1, mb, 256), lambda b, i: (b, i, 0)),
            pl.BlockSpec((1, mb, 1), lambda b, i: (b, i, 0)),
        ),
    )(srcT, cent1, s0row, H0, W_down1, W_red1, Wt2, Wa2)


def _stage2(srcT2, cent2, s1row, H1, W_down2, W_red2, mb=256):
    m = N // 4
    ns = N // 2
    return pl.pallas_call(
        _stage2_body,
        out_shape=jax.ShapeDtypeStruct((B, m, 64), jnp.float32),
        grid=(B, m // mb),
        in_specs=[
            pl.BlockSpec((1, 8, ns), lambda b, i: (b, 0, 0)),
            pl.BlockSpec((1, mb, TABW), lambda b, i: (b, i, 0)),
            pl.BlockSpec((1, 1, ns), lambda b, i: (b, 0, 0)),
            pl.BlockSpec((1, ns, 256), lambda b, i: (b, 0, 0)),
            pl.BlockSpec((256, 256), lambda b, i: (0, 0)),
            pl.BlockSpec((256, 64), lambda b, i: (0, 0)),
        ],
        out_specs=pl.BlockSpec((1, mb, 64), lambda b, i: (b, i, 0)),
    )(srcT2, cent2, s1row, H1, W_down2, W_red2)


def _gate_body(x1_ref, x2_ref, c1w1_ref, c1w2_ref, c2w1_ref, c2w2_ref,
               s1_ref, s2_ref):
    x1u = jnp.repeat(x1_ref[...], 2, axis=1)
    x2u = jnp.repeat(x2_ref[...], 4, axis=1)
    g1 = jnp.mean(x1u, axis=1)
    s1_ref[...] = jax.nn.sigmoid(
        _bdot(jax.nn.relu(_bdot(g1, c1w1_ref[...])), c1w2_ref[...]))
    g2 = jnp.mean(x2u, axis=1)
    s2_ref[...] = jax.nn.sigmoid(
        _bdot(jax.nn.relu(_bdot(g2, c2w1_ref[...])), c2w2_ref[...]))


def _gates(x1, x2, C1_W1, C1_W2, C2_W1, C2_W2):
    full = lambda *shape: pl.BlockSpec(shape, lambda: tuple(0 for _ in shape))
    return pl.pallas_call(
        _gate_body,
        out_shape=(jax.ShapeDtypeStruct((B, 64), jnp.float32),
                   jax.ShapeDtypeStruct((B, 64), jnp.float32)),
        grid=(),
        in_specs=[full(B, N // 2, 64), full(B, N // 4, 64),
                  full(64, 64), full(64, 64), full(64, 64), full(64, 64)],
        out_specs=(full(B, 64), full(B, 64)),
    )(x1, x2, C1_W1, C1_W2, C2_W1, C2_W2)


def _epi_y_body(x0_ref, x1_ref, x2_ref, s1_ref, s2_ref, wo1_ref,
                y_ref, ysum_ref, ysq_ref):
    b = pl.program_id(0)
    i = pl.program_id(1)
    x0 = x0_ref[0]
    x1u = jnp.repeat(x1_ref[0], 2, axis=0)
    x2u = jnp.repeat(x2_ref[0], 4, axis=0)
    sig1 = s1_ref[pl.ds(b, 1), :]
    sig2 = s2_ref[pl.ds(b, 1), :]
    x1e = x0 * sig1 + x1u
    x0e = x1e * sig2 + x2u
    fused = jnp.concatenate([x0e, x1u, x2u], axis=1)
    y = _bdot(fused, wo1_ref[...])
    y_ref[0] = y

    @pl.when(jnp.logical_and(b == 0, i == 0))
    def _():
        ysum_ref[...] = jnp.zeros_like(ysum_ref)
        ysq_ref[...] = jnp.zeros_like(ysq_ref)

    ysum_ref[...] += jnp.sum(y, axis=0, keepdims=True)
    ysq_ref[...] += jnp.sum(y * y, axis=0, keepdims=True)


def _epi_y(x0, x1, x2, sig1, sig2, W_out1, nb=2048):
    full = lambda *shape: pl.BlockSpec(shape, lambda b, i: tuple(0 for _ in shape))
    return pl.pallas_call(
        _epi_y_body,
        out_shape=(jax.ShapeDtypeStruct((B, N, 256), jnp.float32),
                   jax.ShapeDtypeStruct((1, 256), jnp.float32),
                   jax.ShapeDtypeStruct((1, 256), jnp.float32)),
        grid=(B, N // nb),
        in_specs=[
            pl.BlockSpec((1, nb, 64), lambda b, i: (b, i, 0)),
            pl.BlockSpec((1, nb // 2, 64), lambda b, i: (b, i, 0)),
            pl.BlockSpec((1, nb // 4, 64), lambda b, i: (b, i, 0)),
            full(B, 64), full(B, 64), full(192, 256),
        ],
        out_specs=(
            pl.BlockSpec((1, nb, 256), lambda b, i: (b, i, 0)),
            pl.BlockSpec((1, 256), lambda b, i: (0, 0)),
            pl.BlockSpec((1, 256), lambda b, i: (0, 0)),
        ),
    )(x0, x1, x2, sig1, sig2, W_out1)


def _epi_out_body(y_ref, ysum_ref, ysq_ref, gam_ref, bet_ref,
                  wo2_ref, bo2_ref, out_ref):
    cnt = jnp.float32(B * N)
    mu = ysum_ref[...] / cnt
    var = ysq_ref[...] / cnt - mu * mu
    y = y_ref[0]
    h = (y - mu) / jnp.sqrt(var + 1e-5) * gam_ref[...] + bet_ref[...]
    h = jax.nn.relu(h)
    out_ref[0] = _bdot(h, wo2_ref[...]) + bo2_ref[...]


def _epi_out(y, ysum, ysq, bn_gamma, bn_beta, W_out2, b_out2, nb=2048):
    nc = W_out2.shape[1]
    full = lambda *shape: pl.BlockSpec(shape, lambda b, i: tuple(0 for _ in shape))
    return pl.pallas_call(
        _epi_out_body,
        out_shape=jax.ShapeDtypeStruct((B, N, nc), jnp.float32),
        grid=(B, N // nb),
        in_specs=[
            pl.BlockSpec((1, nb, 256), lambda b, i: (b, i, 0)),
            full(1, 256), full(1, 256), full(1, 256), full(1, 256),
            full(256, nc), full(1, nc),
        ],
        out_specs=pl.BlockSpec((1, nb, nc), lambda b, i: (b, i, 0)),
    )(y, ysum, ysq, bn_gamma.reshape(1, 256), bn_beta.reshape(1, 256),
      W_out2, b_out2.reshape(1, nc))


def _epilogue(x0, x1, x2, C1_W1, C1_W2, C2_W1, C2_W2,
              W_out1, bn_gamma, bn_beta, W_out2, b_out2):
    sig1, sig2 = _gates(x1, x2, C1_W1, C1_W2, C2_W1, C2_W2)
    y, ysum, ysq = _epi_y(x0, x1, x2, sig1, sig2, W_out1)
    return _epi_out(y, ysum, ysq, bn_gamma, bn_beta, W_out2, b_out2)


def kernel(points, center_idx1, center_idx2, W_in, Wt1, Wa1, W_down1, W_red1,
           Wt2, Wa2, W_down2, W_red2, C1_W1, C1_W2, C2_W1, C2_W2,
           W_out1, bn_gamma, bn_beta, W_out2, b_out2):
    coords = points[:, :, :3]
    x0, H0, S0col = _prologue(coords, W_in, Wt1, Wa1)

    srcT = jnp.pad(jnp.swapaxes(coords, 1, 2), ((0, 0), (0, 5), (0, 0)))
    s0row = jnp.swapaxes(S0col, 1, 2)

    cent1 = _gather_centers(coords, S0col, center_idx1)
    x1, H1, S1col = _stage1(srcT, cent1, s0row, H0,
                            W_down1, W_red1, Wt2, Wa2)

    srcT2 = srcT[:, :, :N // 2]
    s1row = jnp.swapaxes(S1col, 1, 2)
    cent2 = _gather_centers(coords[:, :N // 2], S1col, center_idx2)
    x2 = _stage2(srcT2, cent2, s1row, H1, W_down2, W_red2)

    return _epilogue(x0, x1, x2, C1_W1, C1_W2, C2_W1, C2_W2,
                     W_out1, bn_gamma, bn_beta, W_out2, b_out2)


# speedup vs baseline: 15.7946x; 1.0156x over previous
import functools

import jax
import jax.numpy as jnp
from jax.experimental import pallas as pl
from jax.experimental.pallas import tpu as pltpu
from jax.experimental.pallas import tpu_sc as plsc

B = 2
N = 8192
KNN = 32
TABW = 128

def _dot(a, b):
    ah = a.astype(jnp.bfloat16)
    al = (a - ah.astype(jnp.float32)).astype(jnp.bfloat16)
    bh = b.astype(jnp.bfloat16)
    bl = (b - bh.astype(jnp.float32)).astype(jnp.bfloat16)
    mm = lambda x, y: jnp.dot(x, y, preferred_element_type=jnp.float32)
    return mm(ah, bh) + (mm(ah, bl) + mm(al, bh))


def _bdot(a, b):
    return jnp.dot(a.astype(jnp.bfloat16), b.astype(jnp.bfloat16),
                   preferred_element_type=jnp.float32)


def _prologue_body(coords_ref, win_ref, wt1_ref, wa1_ref,
                   x0_ref, h0_ref, s0_ref):
    coords = coords_ref[0]
    x0 = _bdot(coords, win_ref[...])
    h0 = jax.nn.relu(_bdot(x0, wt1_ref[...]))
    s0 = _bdot(h0, wa1_ref[...])
    x0_ref[0] = x0
    h0_ref[0] = h0
    s0_ref[0] = s0


def _prologue(coords, W_in, Wt1, Wa1):
    return pl.pallas_call(
        _prologue_body,
        out_shape=(
            jax.ShapeDtypeStruct((B, N, 64), jnp.float32),
            jax.ShapeDtypeStruct((B, N, 128), jnp.float32),
            jax.ShapeDtypeStruct((B, N, 1), jnp.float32),
        ),
        grid=(B,),
        in_specs=[
            pl.BlockSpec((1, N, 3), lambda b: (b, 0, 0)),
            pl.BlockSpec((3, 64), lambda b: (0, 0)),
            pl.BlockSpec((64, 128), lambda b: (0, 0)),
            pl.BlockSpec((128, 1), lambda b: (0, 0)),
        ],
        out_specs=(
            pl.BlockSpec((1, N, 64), lambda b: (b, 0, 0)),
            pl.BlockSpec((1, N, 128), lambda b: (b, 0, 0)),
            pl.BlockSpec((1, N, 1), lambda b: (b, 0, 0)),
        ),
    )(coords, W_in, Wt1, Wa1)


def _sc_gather_rows(table, idx):
    num = idx.shape[0]
    win = 128
    idx2 = idx.reshape(1, num)
    mesh = plsc.VectorSubcoreMesh(core_axis_name="c", subcore_axis_name="s")

    @functools.partial(
        pl.kernel,
        out_type=jax.ShapeDtypeStruct((num, TABW), jnp.float32),
        mesh=mesh,
    )
    def k(tab_hbm, i_hbm, o_hbm):
        def body(i_vmem, o_vmem):
            pltpu.sync_copy(tab_hbm.at[i_vmem.at[0]], o_vmem)

        pltpu.emit_pipeline(
            body,
            grid=(num // win,),
            in_specs=[pl.BlockSpec((1, win), index_map=lambda i: (0, i))],
            out_specs=[pl.BlockSpec((win, TABW), index_map=lambda i: (i, 0))],
            core_axis_name=("c", "s"),
            dimension_semantics=(pltpu.PARALLEL,),
        )(i_hbm, o_hbm)

    return k(table, idx2)


def _gather_centers(coords_part, s_col, center_idx):
    ns = coords_part.shape[1]
    m = center_idx.shape[1]
    pad = jnp.zeros((B, ns, TABW - 4), jnp.float32)
    table = jnp.concatenate([coords_part, s_col, pad], axis=-1)
    table = table.reshape(B * ns, TABW)
    idx = (center_idx.astype(jnp.int32)
           + (jnp.arange(B, dtype=jnp.int32) * ns)[:, None]).reshape(-1)
    out = _sc_gather_rows(table, idx)
    return out.reshape(B, m, TABW)


def _knn_weights(bits, cent, s_row, r2):
    mb, ns = bits.shape

    def srch(i, acc):
        cand = acc | jax.lax.shift_left(jnp.int32(1), jnp.int32(30) - i)
        cnt = jnp.sum((bits < cand).astype(jnp.int32), axis=1, keepdims=True)
        return jnp.where(cnt < KNN, cand, acc)

    tstar = jax.lax.fori_loop(0, 31, srch, jnp.zeros((mb, 1), jnp.int32))

    r2bits = jnp.float32(r2).view(jnp.int32)
    sel = bits <= tstar
    val = sel & (bits <= r2bits)

    neg = jnp.float32(-3.4e38)
    mx = jnp.max(jnp.where(val, s_row, neg), axis=1, keepdims=True)
    ew = jnp.where(val, jnp.exp(s_row - mx), 0.0)
    sume = jnp.sum(ew, axis=1, keepdims=True)
    cnt_val = jnp.sum(val.astype(jnp.float32), axis=1, keepdims=True)
    nv = jnp.maximum(jnp.float32(KNN) - cnt_val, 0.0)

    s0c = cent[:, 3:4]
    ec = jnp.exp(s0c - mx) * nv
    denom = sume + ec
    invd = 1.0 / denom

    iota = jax.lax.broadcasted_iota(jnp.int32, (mb, ns), 1)
    cidx = jnp.min(jnp.where(bits == 0, iota, jnp.int32(ns)), axis=1,
                   keepdims=True)
    w = ew * invd + jnp.where(iota == cidx, ec * invd, 0.0)
    return w


def _d2bits(cent, src_t):
    cx = cent[:, 0:1]
    cy = cent[:, 1:2]
    cz = cent[:, 2:3]
    dx = src_t[0:1, :] - cx
    dy = src_t[1:2, :] - cy
    dz = src_t[2:3, :] - cz
    d2 = dx * dx + dy * dy + dz * dz
    return d2.view(jnp.int32)


def _stage1_body(srcT_ref, cent_ref, s0row_ref, h0_ref,
                 wd_ref, wr_ref, wt2_ref, wa2_ref,
                 x1_ref, h1_ref, s1_ref):
    cent = cent_ref[0]
    bits = _d2bits(cent, srcT_ref[0])
    s_row = s0row_ref[0]
    w = _knn_weights(bits, cent, s_row, 0.3 ** 2)
    agg = _dot(w, h0_ref[0])
    x1 = _bdot(_bdot(agg, wd_ref[...]), wr_ref[...])
    h1 = jax.nn.relu(_bdot(x1, wt2_ref[...]))
    s1 = _bdot(h1, wa2_ref[...])
    x1_ref[0] = x1
    h1_ref[0] = h1
    s1_ref[0] = s1


def _stage2_body(srcT_ref, cent_ref, s1row_ref, h1_ref,
                 wd_ref, wr_ref, x2_ref):
    cent = cent_ref[0]
    bits = _d2bits(cent, srcT_ref[0])
    s_row = s1row_ref[0]
    w = _knn_weights(bits, cent, s_row, 0.5 ** 2)
    agg = _dot(w, h1_ref[0])
    x2_ref[0] = _bdot(_bdot(agg, wd_ref[...]), wr_ref[...])


def _stage1(srcT, cent1, s0row, H0, W_down1, W_red1, Wt2, Wa2, mb=256):
    m = N // 2
    return pl.pallas_call(
        _stage1_body,
        out_shape=(
            jax.ShapeDtypeStruct((B, m, 64), jnp.float32),
            jax.ShapeDtypeStruct((B, m, 256), jnp.float32),
            jax.ShapeDtypeStruct((B, m, 1), jnp.float32),
        ),
        grid=(B, m // mb),
        in_specs=[
            pl.BlockSpec((1, 8, N), lambda b, i: (b, 0, 0)),
            pl.BlockSpec((1, mb, TABW), lambda b, i: (b, i, 0)),
            pl.BlockSpec((1, 1, N), lambda b, i: (b, 0, 0)),
            pl.BlockSpec((1, N, 128), lambda b, i: (b, 0, 0)),
            pl.BlockSpec((128, 128), lambda b, i: (0, 0)),
            pl.BlockSpec((128, 64), lambda b, i: (0, 0)),
            pl.BlockSpec((64, 256), lambda b, i: (0, 0)),
            pl.BlockSpec((256, 1), lambda b, i: (0, 0)),
        ],
        out_specs=(
            pl.BlockSpec((1, mb, 64), lambda b, i: (b, i, 0)),
            pl.BlockSpec((1, mb, 256), lambda b, i: (b, i, 0)),
            pl.BlockSpec((1, mb, 1), lambda b, i: (b, i, 0)),
        ),
    )(srcT, cent1, s0row, H0, W_down1, W_red1, Wt2, Wa2)


def _stage2(srcT2, cent2, s1row, H1, W_down2, W_red2, mb=512):
    m = N // 4
    ns = N // 2
    return pl.pallas_call(
        _stage2_body,
        out_shape=jax.ShapeDtypeStruct((B, m, 64), jnp.float32),
        grid=(B, m // mb),
        in_specs=[
            pl.BlockSpec((1, 8, ns), lambda b, i: (b, 0, 0)),
            pl.BlockSpec((1, mb, TABW), lambda b, i: (b, i, 0)),
            pl.BlockSpec((1, 1, ns), lambda b, i: (b, 0, 0)),
            pl.BlockSpec((1, ns, 256), lambda b, i: (b, 0, 0)),
            pl.BlockSpec((256, 256), lambda b, i: (0, 0)),
            pl.BlockSpec((256, 64), lambda b, i: (0, 0)),
        ],
        out_specs=pl.BlockSpec((1, mb, 64), lambda b, i: (b, i, 0)),
    )(srcT2, cent2, s1row, H1, W_down2, W_red2)


def _gate_body(x1_ref, x2_ref, c1w1_ref, c1w2_ref, c2w1_ref, c2w2_ref,
               s1_ref, s2_ref):
    x1u = jnp.repeat(x1_ref[...], 2, axis=1)
    x2u = jnp.repeat(x2_ref[...], 4, axis=1)
    g1 = jnp.mean(x1u, axis=1)
    s1_ref[...] = jax.nn.sigmoid(
        _bdot(jax.nn.relu(_bdot(g1, c1w1_ref[...])), c1w2_ref[...]))
    g2 = jnp.mean(x2u, axis=1)
    s2_ref[...] = jax.nn.sigmoid(
        _bdot(jax.nn.relu(_bdot(g2, c2w1_ref[...])), c2w2_ref[...]))


def _gates(x1, x2, C1_W1, C1_W2, C2_W1, C2_W2):
    full = lambda *shape: pl.BlockSpec(shape, lambda: tuple(0 for _ in shape))
    return pl.pallas_call(
        _gate_body,
        out_shape=(jax.ShapeDtypeStruct((B, 64), jnp.float32),
                   jax.ShapeDtypeStruct((B, 64), jnp.float32)),
        grid=(),
        in_specs=[full(B, N // 2, 64), full(B, N // 4, 64),
                  full(64, 64), full(64, 64), full(64, 64), full(64, 64)],
        out_specs=(full(B, 64), full(B, 64)),
    )(x1, x2, C1_W1, C1_W2, C2_W1, C2_W2)


def _epi_y_body(x0_ref, x1_ref, x2_ref, s1_ref, s2_ref, wo1_ref,
                y_ref, ysum_ref, ysq_ref):
    b = pl.program_id(0)
    i = pl.program_id(1)
    x0 = x0_ref[0]
    x1u = jnp.repeat(x1_ref[0], 2, axis=0)
    x2u = jnp.repeat(x2_ref[0], 4, axis=0)
    sig1 = s1_ref[pl.ds(b, 1), :]
    sig2 = s2_ref[pl.ds(b, 1), :]
    x1e = x0 * sig1 + x1u
    x0e = x1e * sig2 + x2u
    fused = jnp.concatenate([x0e, x1u, x2u], axis=1)
    y = _bdot(fused, wo1_ref[...])
    y_ref[0] = y

    @pl.when(jnp.logical_and(b == 0, i == 0))
    def _():
        ysum_ref[...] = jnp.zeros_like(ysum_ref)
        ysq_ref[...] = jnp.zeros_like(ysq_ref)

    ysum_ref[...] += jnp.sum(y, axis=0, keepdims=True)
    ysq_ref[...] += jnp.sum(y * y, axis=0, keepdims=True)


def _epi_y(x0, x1, x2, sig1, sig2, W_out1, nb=2048):
    full = lambda *shape: pl.BlockSpec(shape, lambda b, i: tuple(0 for _ in shape))
    return pl.pallas_call(
        _epi_y_body,
        out_shape=(jax.ShapeDtypeStruct((B, N, 256), jnp.float32),
                   jax.ShapeDtypeStruct((1, 256), jnp.float32),
                   jax.ShapeDtypeStruct((1, 256), jnp.float32)),
        grid=(B, N // nb),
        in_specs=[
            pl.BlockSpec((1, nb, 64), lambda b, i: (b, i, 0)),
            pl.BlockSpec((1, nb // 2, 64), lambda b, i: (b, i, 0)),
            pl.BlockSpec((1, nb // 4, 64), lambda b, i: (b, i, 0)),
            full(B, 64), full(B, 64), full(192, 256),
        ],
        out_specs=(
            pl.BlockSpec((1, nb, 256), lambda b, i: (b, i, 0)),
            pl.BlockSpec((1, 256), lambda b, i: (0, 0)),
            pl.BlockSpec((1, 256), lambda b, i: (0, 0)),
        ),
    )(x0, x1, x2, sig1, sig2, W_out1)


def _epi_out_body(y_ref, ysum_ref, ysq_ref, gam_ref, bet_ref,
                  wo2_ref, bo2_ref, out_ref):
    cnt = jnp.float32(B * N)
    mu = ysum_ref[...] / cnt
    var = ysq_ref[...] / cnt - mu * mu
    y = y_ref[0]
    h = (y - mu) / jnp.sqrt(var + 1e-5) * gam_ref[...] + bet_ref[...]
    h = jax.nn.relu(h)
    out_ref[0] = _bdot(h, wo2_ref[...]) + bo2_ref[...]


def _epi_out(y, ysum, ysq, bn_gamma, bn_beta, W_out2, b_out2, nb=2048):
    nc = W_out2.shape[1]
    full = lambda *shape: pl.BlockSpec(shape, lambda b, i: tuple(0 for _ in shape))
    return pl.pallas_call(
        _epi_out_body,
        out_shape=jax.ShapeDtypeStruct((B, N, nc), jnp.float32),
        grid=(B, N // nb),
        in_specs=[
            pl.BlockSpec((1, nb, 256), lambda b, i: (b, i, 0)),
            full(1, 256), full(1, 256), full(1, 256), full(1, 256),
            full(256, nc), full(1, nc),
        ],
        out_specs=pl.BlockSpec((1, nb, nc), lambda b, i: (b, i, 0)),
    )(y, ysum, ysq, bn_gamma.reshape(1, 256), bn_beta.reshape(1, 256),
      W_out2, b_out2.reshape(1, nc))


def _epilogue(x0, x1, x2, C1_W1, C1_W2, C2_W1, C2_W2,
              W_out1, bn_gamma, bn_beta, W_out2, b_out2):
    sig1, sig2 = _gates(x1, x2, C1_W1, C1_W2, C2_W1, C2_W2)
    y, ysum, ysq = _epi_y(x0, x1, x2, sig1, sig2, W_out1)
    return _epi_out(y, ysum, ysq, bn_gamma, bn_beta, W_out2, b_out2)


def kernel(points, center_idx1, center_idx2, W_in, Wt1, Wa1, W_down1, W_red1,
           Wt2, Wa2, W_down2, W_red2, C1_W1, C1_W2, C2_W1, C2_W2,
           W_out1, bn_gamma, bn_beta, W_out2, b_out2):
    coords = points[:, :, :3]
    x0, H0, S0col = _prologue(coords, W_in, Wt1, Wa1)

    srcT = jnp.pad(jnp.swapaxes(coords, 1, 2), ((0, 0), (0, 5), (0, 0)))
    s0row = jnp.swapaxes(S0col, 1, 2)

    cent1 = _gather_centers(coords, S0col, center_idx1)
    x1, H1, S1col = _stage1(srcT, cent1, s0row, H0,
                            W_down1, W_red1, Wt2, Wa2)

    srcT2 = srcT[:, :, :N // 2]
    s1row = jnp.swapaxes(S1col, 1, 2)
    cent2 = _gather_centers(coords[:, :N // 2], S1col, center_idx2)
    x2 = _stage2(srcT2, cent2, s1row, H1, W_down2, W_red2)

    return _epilogue(x0, x1, x2, C1_W1, C1_W2, C2_W1, C2_W2,
                     W_out1, bn_gamma, bn_beta, W_out2, b_out2)


# speedup vs baseline: 16.5462x; 1.0476x over previous
import functools

import jax
import jax.numpy as jnp
from jax.experimental import pallas as pl
from jax.experimental.pallas import tpu as pltpu
from jax.experimental.pallas import tpu_sc as plsc

B = 2
N = 8192
KNN = 32


def _dot(a, b):
    ah = a.astype(jnp.bfloat16)
    al = (a - ah.astype(jnp.float32)).astype(jnp.bfloat16)
    bh = b.astype(jnp.bfloat16)
    bl = (b - bh.astype(jnp.float32)).astype(jnp.bfloat16)
    mm = lambda x, y: jnp.dot(x, y, preferred_element_type=jnp.float32)
    return mm(ah, bh) + (mm(ah, bl) + mm(al, bh))


def _bdot(a, b):
    return jnp.dot(a.astype(jnp.bfloat16), b.astype(jnp.bfloat16),
                   preferred_element_type=jnp.float32)


def _prologue_body(coords_ref, win_ref, wt1_ref, wa1_ref,
                   x0_ref, h0_ref, s0_ref):
    coords = coords_ref[0]
    x0 = _bdot(coords, win_ref[...])
    h0 = jax.nn.relu(_bdot(x0, wt1_ref[...]))
    s0 = _bdot(h0, wa1_ref[...])
    x0_ref[0] = x0
    h0_ref[0] = h0
    s0_ref[0] = s0


def _prologue(coords, W_in, Wt1, Wa1):
    return pl.pallas_call(
        _prologue_body,
        out_shape=(
            jax.ShapeDtypeStruct((B, N, 64), jnp.float32),
            jax.ShapeDtypeStruct((B, N, 128), jnp.float32),
            jax.ShapeDtypeStruct((B, N, 1), jnp.float32),
        ),
        grid=(B,),
        in_specs=[
            pl.BlockSpec((1, N, 3), lambda b: (b, 0, 0)),
            pl.BlockSpec((3, 64), lambda b: (0, 0)),
            pl.BlockSpec((64, 128), lambda b: (0, 0)),
            pl.BlockSpec((128, 1), lambda b: (0, 0)),
        ],
        out_specs=(
            pl.BlockSpec((1, N, 64), lambda b: (b, 0, 0)),
            pl.BlockSpec((1, N, 128), lambda b: (b, 0, 0)),
            pl.BlockSpec((1, N, 1), lambda b: (b, 0, 0)),
        ),
    )(coords, W_in, Wt1, Wa1)


def _sc_gather_rows(table, idx):
    num = idx.shape[0]
    tabw = table.shape[1]
    win = 128
    idx2 = idx.reshape(1, num)
    mesh = plsc.VectorSubcoreMesh(core_axis_name="c", subcore_axis_name="s")

    @functools.partial(
        pl.kernel,
        out_type=jax.ShapeDtypeStruct((num, tabw), jnp.float32),
        mesh=mesh,
    )
    def k(tab_hbm, i_hbm, o_hbm):
        def body(i_vmem, o_vmem):
            pltpu.sync_copy(tab_hbm.at[i_vmem.at[0]], o_vmem)

        pltpu.emit_pipeline(
            body,
            grid=(num // win,),
            in_specs=[pl.BlockSpec((1, win), index_map=lambda i: (0, i))],
            out_specs=[pl.BlockSpec((win, tabw), index_map=lambda i: (i, 0))],
            core_axis_name=("c", "s"),
            dimension_semantics=(pltpu.PARALLEL,),
        )(i_hbm, o_hbm)

    return k(table, idx2)


def _gather_centers(coords_part, s_col, h, center_idx):
    ns = coords_part.shape[1]
    m = center_idx.shape[1]
    pad = jnp.zeros((B, ns, 124), jnp.float32)
    table = jnp.concatenate([coords_part, s_col, pad, h], axis=-1)
    tabw = table.shape[2]
    table = table.reshape(B * ns, tabw)
    idx = (center_idx.astype(jnp.int32)
           + (jnp.arange(B, dtype=jnp.int32) * ns)[:, None]).reshape(-1)
    out = _sc_gather_rows(table, idx)
    return out.reshape(B, m, tabw)


def _knn_weights(bits, cent, s_row, r2):
    mb, ns = bits.shape

    def srch(i, acc):
        cand = acc | jax.lax.shift_left(jnp.int32(1), jnp.int32(30) - i)
        cnt = jnp.sum((bits < cand).astype(jnp.int32), axis=1, keepdims=True)
        return jnp.where(cnt < KNN, cand, acc)

    tstar = jax.lax.fori_loop(0, 31, srch, jnp.zeros((mb, 1), jnp.int32))

    r2bits = jnp.float32(r2).view(jnp.int32)
    val = bits <= jnp.minimum(tstar, r2bits)

    neg = jnp.float32(-3.4e38)
    mx = jnp.max(jnp.where(val, s_row, neg), axis=1, keepdims=True)
    ew = jnp.where(val, jnp.exp(s_row - mx), 0.0)
    sume = jnp.sum(ew, axis=1, keepdims=True)
    cnt_val = jnp.sum(val.astype(jnp.float32), axis=1, keepdims=True)
    nv = jnp.maximum(jnp.float32(KNN) - cnt_val, 0.0)

    s0c = cent[:, 3:4]
    ec = jnp.exp(s0c - mx) * nv
    invd = 1.0 / (sume + ec)
    return ew * invd, ec * invd


def _d2bits(cent, src_t):
    cx = cent[:, 0:1]
    cy = cent[:, 1:2]
    cz = cent[:, 2:3]
    dx = src_t[0:1, :] - cx
    dy = src_t[1:2, :] - cy
    dz = src_t[2:3, :] - cz
    d2 = dx * dx + dy * dy + dz * dz
    return d2.view(jnp.int32)


def _stage1_body(srcT_ref, cent_ref, s0row_ref, h0_ref,
                 wd_ref, wr_ref, wt2_ref, wa2_ref,
                 x1_ref, h1_ref, s1_ref):
    cent = cent_ref[0]
    bits = _d2bits(cent, srcT_ref[0])
    s_row = s0row_ref[0]
    w, cadd = _knn_weights(bits, cent, s_row, 0.3 ** 2)
    agg = _dot(w, h0_ref[0]) + cadd * cent[:, 128:256]
    x1 = _bdot(_bdot(agg, wd_ref[...]), wr_ref[...])
    h1 = jax.nn.relu(_bdot(x1, wt2_ref[...]))
    s1 = _bdot(h1, wa2_ref[...])
    x1_ref[0] = x1
    h1_ref[0] = h1
    s1_ref[0] = s1


def _stage2_body(srcT_ref, cent_ref, s1row_ref, h1_ref,
                 wd_ref, wr_ref, x2_ref):
    cent = cent_ref[0]
    bits = _d2bits(cent, srcT_ref[0])
    s_row = s1row_ref[0]
    w, cadd = _knn_weights(bits, cent, s_row, 0.5 ** 2)
    agg = _dot(w, h1_ref[0]) + cadd * cent[:, 128:384]
    x2_ref[0] = _bdot(_bdot(agg, wd_ref[...]), wr_ref[...])


def _stage1(srcT, cent1, s0row, H0, W_down1, W_red1, Wt2, Wa2, mb=256):
    m = N // 2
    return pl.pallas_call(
        _stage1_body,
        out_shape=(
            jax.ShapeDtypeStruct((B, m, 64), jnp.float32),
            jax.ShapeDtypeStruct((B, m, 256), jnp.float32),
            jax.ShapeDtypeStruct((B, m, 1), jnp.float32),
        ),
        grid=(B, m // mb),
        in_specs=[
            pl.BlockSpec((1, 8, N), lambda b, i: (b, 0, 0)),
            pl.BlockSpec((1, mb, 256), lambda b, i: (b, i, 0)),
            pl.BlockSpec((1, 1, N), lambda b, i: (b, 0, 0)),
            pl.BlockSpec((1, N, 128), lambda b, i: (b, 0, 0)),
            pl.BlockSpec((128, 128), lambda b, i: (0, 0)),
            pl.BlockSpec((128, 64), lambda b, i: (0, 0)),
            pl.BlockSpec((64, 256), lambda b, i: (0, 0)),
            pl.BlockSpec((256, 1), lambda b, i: (0, 0)),
        ],
        out_specs=(
            pl.BlockSpec((1, mb, 64), lambda b, i: (b, i, 0)),
            pl.BlockSpec((1, mb, 256), lambda b, i: (b, i, 0)),
            pl.BlockSpec((1, mb, 1), lambda b, i: (b, i, 0)),
        ),
    )(srcT, cent1, s0row, H0, W_down1, W_red1, Wt2, Wa2)


def _stage2(srcT2, cent2, s1row, H1, W_down2, W_red2, mb=512):
    m = N // 4
    ns = N // 2
    return pl.pallas_call(
        _stage2_body,
        out_shape=jax.ShapeDtypeStruct((B, m, 64), jnp.float32),
        grid=(B, m // mb),
        in_specs=[
            pl.BlockSpec((1, 8, ns), lambda b, i: (b, 0, 0)),
            pl.BlockSpec((1, mb, 384), lambda b, i: (b, i, 0)),
            pl.BlockSpec((1, 1, ns), lambda b, i: (b, 0, 0)),
            pl.BlockSpec((1, ns, 256), lambda b, i: (b, 0, 0)),
            pl.BlockSpec((256, 256), lambda b, i: (0, 0)),
            pl.BlockSpec((256, 64), lambda b, i: (0, 0)),
        ],
        out_specs=pl.BlockSpec((1, mb, 64), lambda b, i: (b, i, 0)),
    )(srcT2, cent2, s1row, H1, W_down2, W_red2)


def _gate_body(x1_ref, x2_ref, c1w1_ref, c1w2_ref, c2w1_ref, c2w2_ref,
               s1_ref, s2_ref):
    x1u = jnp.repeat(x1_ref[...], 2, axis=1)
    x2u = jnp.repeat(x2_ref[...], 4, axis=1)
    g1 = jnp.mean(x1u, axis=1)
    s1_ref[...] = jax.nn.sigmoid(
        _bdot(jax.nn.relu(_bdot(g1, c1w1_ref[...])), c1w2_ref[...]))
    g2 = jnp.mean(x2u, axis=1)
    s2_ref[...] = jax.nn.sigmoid(
        _bdot(jax.nn.relu(_bdot(g2, c2w1_ref[...])), c2w2_ref[...]))


def _gates(x1, x2, C1_W1, C1_W2, C2_W1, C2_W2):
    full = lambda *shape: pl.BlockSpec(shape, lambda: tuple(0 for _ in shape))
    return pl.pallas_call(
        _gate_body,
        out_shape=(jax.ShapeDtypeStruct((B, 64), jnp.float32),
                   jax.ShapeDtypeStruct((B, 64), jnp.float32)),
        grid=(),
        in_specs=[full(B, N // 2, 64), full(B, N // 4, 64),
                  full(64, 64), full(64, 64), full(64, 64), full(64, 64)],
        out_specs=(full(B, 64), full(B, 64)),
    )(x1, x2, C1_W1, C1_W2, C2_W1, C2_W2)


def _epi_y_body(x0_ref, x1_ref, x2_ref, s1_ref, s2_ref, wo1_ref,
                y_ref, ysum_ref, ysq_ref):
    b = pl.program_id(0)
    i = pl.program_id(1)
    x0 = x0_ref[0]
    x1u = jnp.repeat(x1_ref[0], 2, axis=0)
    x2u = jnp.repeat(x2_ref[0], 4, axis=0)
    sig1 = s1_ref[pl.ds(b, 1), :]
    sig2 = s2_ref[pl.ds(b, 1), :]
    x1e = x0 * sig1 + x1u
    x0e = x1e * sig2 + x2u
    fused = jnp.concatenate([x0e, x1u, x2u], axis=1)
    y = _bdot(fused, wo1_ref[...])
    y_ref[0] = y

    @pl.when(jnp.logical_and(b == 0, i == 0))
    def _():
        ysum_ref[...] = jnp.zeros_like(ysum_ref)
        ysq_ref[...] = jnp.zeros_like(ysq_ref)

    ysum_ref[...] += jnp.sum(y, axis=0, keepdims=True)
    ysq_ref[...] += jnp.sum(y * y, axis=0, keepdims=True)


def _epi_y(x0, x1, x2, sig1, sig2, W_out1, nb=2048):
    full = lambda *shape: pl.BlockSpec(shape, lambda b, i: tuple(0 for _ in shape))
    return pl.pallas_call(
        _epi_y_body,
        out_shape=(jax.ShapeDtypeStruct((B, N, 256), jnp.float32),
                   jax.ShapeDtypeStruct((1, 256), jnp.float32),
                   jax.ShapeDtypeStruct((1, 256), jnp.float32)),
        grid=(B, N // nb),
        in_specs=[
            pl.BlockSpec((1, nb, 64), lambda b, i: (b, i, 0)),
            pl.BlockSpec((1, nb // 2, 64), lambda b, i: (b, i, 0)),
            pl.BlockSpec((1, nb // 4, 64), lambda b, i: (b, i, 0)),
            full(B, 64), full(B, 64), full(192, 256),
        ],
        out_specs=(
            pl.BlockSpec((1, nb, 256), lambda b, i: (b, i, 0)),
            pl.BlockSpec((1, 256), lambda b, i: (0, 0)),
            pl.BlockSpec((1, 256), lambda b, i: (0, 0)),
        ),
    )(x0, x1, x2, sig1, sig2, W_out1)


def _epi_out_body(y_ref, ysum_ref, ysq_ref, gam_ref, bet_ref,
                  wo2_ref, bo2_ref, out_ref):
    cnt = jnp.float32(B * N)
    mu = ysum_ref[...] / cnt
    var = ysq_ref[...] / cnt - mu * mu
    y = y_ref[0]
    h = (y - mu) / jnp.sqrt(var + 1e-5) * gam_ref[...] + bet_ref[...]
    h = jax.nn.relu(h)
    out_ref[0] = _bdot(h, wo2_ref[...]) + bo2_ref[...]


def _epi_out(y, ysum, ysq, bn_gamma, bn_beta, W_out2, b_out2, nb=2048):
    nc = W_out2.shape[1]
    full = lambda *shape: pl.BlockSpec(shape, lambda b, i: tuple(0 for _ in shape))
    return pl.pallas_call(
        _epi_out_body,
        out_shape=jax.ShapeDtypeStruct((B, N, nc), jnp.float32),
        grid=(B, N // nb),
        in_specs=[
            pl.BlockSpec((1, nb, 256), lambda b, i: (b, i, 0)),
            full(1, 256), full(1, 256), full(1, 256), full(1, 256),
            full(256, nc), full(1, nc),
        ],
        out_specs=pl.BlockSpec((1, nb, nc), lambda b, i: (b, i, 0)),
    )(y, ysum, ysq, bn_gamma.reshape(1, 256), bn_beta.reshape(1, 256),
      W_out2, b_out2.reshape(1, nc))


def _epilogue(x0, x1, x2, C1_W1, C1_W2, C2_W1, C2_W2,
              W_out1, bn_gamma, bn_beta, W_out2, b_out2):
    sig1, sig2 = _gates(x1, x2, C1_W1, C1_W2, C2_W1, C2_W2)
    y, ysum, ysq = _epi_y(x0, x1, x2, sig1, sig2, W_out1)
    return _epi_out(y, ysum, ysq, bn_gamma, bn_beta, W_out2, b_out2)


def kernel(points, center_idx1, center_idx2, W_in, Wt1, Wa1, W_down1, W_red1,
           Wt2, Wa2, W_down2, W_red2, C1_W1, C1_W2, C2_W1, C2_W2,
           W_out1, bn_gamma, bn_beta, W_out2, b_out2):
    coords = points[:, :, :3]
    x0, H0, S0col = _prologue(coords, W_in, Wt1, Wa1)

    srcT = jnp.pad(jnp.swapaxes(coords, 1, 2), ((0, 0), (0, 5), (0, 0)))
    s0row = jnp.swapaxes(S0col, 1, 2)

    cent1 = _gather_centers(coords, S0col, H0, center_idx1)
    x1, H1, S1col = _stage1(srcT, cent1, s0row, H0,
                            W_down1, W_red1, Wt2, Wa2)

    srcT2 = srcT[:, :, :N // 2]
    s1row = jnp.swapaxes(S1col, 1, 2)
    cent2 = _gather_centers(coords[:, :N // 2], S1col, H1, center_idx2)
    x2 = _stage2(srcT2, cent2, s1row, H1, W_down2, W_red2)

    return _epilogue(x0, x1, x2, C1_W1, C1_W2, C2_W1, C2_W2,
                     W_out1, bn_gamma, bn_beta, W_out2, b_out2)


# speedup vs baseline: 17.3677x; 1.0496x over previous
import functools

import jax
import jax.numpy as jnp
from jax.experimental import pallas as pl
from jax.experimental.pallas import tpu as pltpu
from jax.experimental.pallas import tpu_sc as plsc

B = 2
N = 8192
KNN = 32


def _split_cat(h):
    hh = h.astype(jnp.bfloat16)
    hl = (h - hh.astype(jnp.float32)).astype(jnp.bfloat16)
    return jnp.concatenate([hh, hl], axis=-1)


def _aggdot(w, hcat):
    c = hcat.shape[1] // 2
    q = jnp.dot(w.astype(jnp.bfloat16), hcat,
                preferred_element_type=jnp.float32)
    return q[:, :c] + q[:, c:]


def _bdot(a, b):
    return jnp.dot(a.astype(jnp.bfloat16), b.astype(jnp.bfloat16),
                   preferred_element_type=jnp.float32)


def _prologue_body(coords_ref, win_ref, wt1_ref, wa1_ref,
                   x0_ref, h0_ref, s0_ref, hc_ref):
    coords = coords_ref[0]
    x0 = _bdot(coords, win_ref[...])
    h0 = jax.nn.relu(_bdot(x0, wt1_ref[...]))
    s0 = _bdot(h0, wa1_ref[...])
    x0_ref[0] = x0
    h0_ref[0] = h0
    s0_ref[0] = s0
    hc_ref[0] = _split_cat(h0)


def _prologue(coords, W_in, Wt1, Wa1):
    return pl.pallas_call(
        _prologue_body,
        out_shape=(
            jax.ShapeDtypeStruct((B, N, 64), jnp.float32),
            jax.ShapeDtypeStruct((B, N, 128), jnp.float32),
            jax.ShapeDtypeStruct((B, N, 1), jnp.float32),
            jax.ShapeDtypeStruct((B, N, 256), jnp.bfloat16),
        ),
        grid=(B,),
        in_specs=[
            pl.BlockSpec((1, N, 3), lambda b: (b, 0, 0)),
            pl.BlockSpec((3, 64), lambda b: (0, 0)),
            pl.BlockSpec((64, 128), lambda b: (0, 0)),
            pl.BlockSpec((128, 1), lambda b: (0, 0)),
        ],
        out_specs=(
            pl.BlockSpec((1, N, 64), lambda b: (b, 0, 0)),
            pl.BlockSpec((1, N, 128), lambda b: (b, 0, 0)),
            pl.BlockSpec((1, N, 1), lambda b: (b, 0, 0)),
            pl.BlockSpec((1, N, 256), lambda b: (b, 0, 0)),
        ),
    )(coords, W_in, Wt1, Wa1)


def _sc_gather_rows(table, idx):
    num = idx.shape[0]
    tabw = table.shape[1]
    win = 128
    idx2 = idx.reshape(1, num)
    mesh = plsc.VectorSubcoreMesh(core_axis_name="c", subcore_axis_name="s")

    @functools.partial(
        pl.kernel,
        out_type=jax.ShapeDtypeStruct((num, tabw), jnp.float32),
        mesh=mesh,
    )
    def k(tab_hbm, i_hbm, o_hbm):
        def body(i_vmem, o_vmem):
            pltpu.sync_copy(tab_hbm.at[i_vmem.at[0]], o_vmem)

        pltpu.emit_pipeline(
            body,
            grid=(num // win,),
            in_specs=[pl.BlockSpec((1, win), index_map=lambda i: (0, i))],
            out_specs=[pl.BlockSpec((win, tabw), index_map=lambda i: (i, 0))],
            core_axis_name=("c", "s"),
            dimension_semantics=(pltpu.PARALLEL,),
        )(i_hbm, o_hbm)

    return k(table, idx2)


def _gather_centers(coords_part, s_col, h, center_idx):
    ns = coords_part.shape[1]
    m = center_idx.shape[1]
    pad = jnp.zeros((B, ns, 124), jnp.float32)
    table = jnp.concatenate([coords_part, s_col, pad, h], axis=-1)
    tabw = table.shape[2]
    table = table.reshape(B * ns, tabw)
    idx = (center_idx.astype(jnp.int32)
           + (jnp.arange(B, dtype=jnp.int32) * ns)[:, None]).reshape(-1)
    out = _sc_gather_rows(table, idx)
    return out.reshape(B, m, tabw)


def _knn_weights(bits, cent, s_row, r2):
    mb, ns = bits.shape

    def srch(i, acc):
        cand = acc | jax.lax.shift_left(jnp.int32(1), jnp.int32(30) - i)
        cnt = jnp.sum((bits < cand).astype(jnp.int32), axis=1, keepdims=True)
        return jnp.where(cnt < KNN, cand, acc)

    tstar = jax.lax.fori_loop(0, 31, srch, jnp.zeros((mb, 1), jnp.int32))

    r2bits = jnp.float32(r2).view(jnp.int32)
    val = bits <= jnp.minimum(tstar, r2bits)

    neg = jnp.float32(-3.4e38)
    mx = jnp.max(jnp.where(val, s_row, neg), axis=1, keepdims=True)
    ew = jnp.where(val, jnp.exp(s_row - mx), 0.0)
    sume = jnp.sum(ew, axis=1, keepdims=True)
    cnt_val = jnp.sum(val.astype(jnp.float32), axis=1, keepdims=True)
    nv = jnp.maximum(jnp.float32(KNN) - cnt_val, 0.0)

    s0c = cent[:, 3:4]
    ec = jnp.exp(s0c - mx) * nv
    invd = 1.0 / (sume + ec)
    return ew * invd, ec * invd


def _d2bits(cent, src_t):
    cx = cent[:, 0:1]
    cy = cent[:, 1:2]
    cz = cent[:, 2:3]
    dx = src_t[0:1, :] - cx
    dy = src_t[1:2, :] - cy
    dz = src_t[2:3, :] - cz
    d2 = dx * dx + dy * dy + dz * dz
    return d2.view(jnp.int32)


def _stage1_body(srcT_ref, cent_ref, s0row_ref, hc_ref,
                 wd_ref, wr_ref, wt2_ref, wa2_ref,
                 x1_ref, h1_ref, s1_ref, h1c_ref):
    cent = cent_ref[0]
    bits = _d2bits(cent, srcT_ref[0])
    s_row = s0row_ref[0]
    w, cadd = _knn_weights(bits, cent, s_row, 0.3 ** 2)
    agg = _aggdot(w, hc_ref[0]) + cadd * cent[:, 128:256]
    x1 = _bdot(_bdot(agg, wd_ref[...]), wr_ref[...])
    h1 = jax.nn.relu(_bdot(x1, wt2_ref[...]))
    s1 = _bdot(h1, wa2_ref[...])
    x1_ref[0] = x1
    h1_ref[0] = h1
    s1_ref[0] = s1
    h1c_ref[0] = _split_cat(h1)


def _stage2_body(srcT_ref, cent_ref, s1row_ref, h1c_ref,
                 wd_ref, wr_ref, x2_ref):
    cent = cent_ref[0]
    bits = _d2bits(cent, srcT_ref[0])
    s_row = s1row_ref[0]
    w, cadd = _knn_weights(bits, cent, s_row, 0.5 ** 2)
    agg = _aggdot(w, h1c_ref[0]) + cadd * cent[:, 128:384]
    x2_ref[0] = _bdot(_bdot(agg, wd_ref[...]), wr_ref[...])


def _stage1(srcT, cent1, s0row, H0cat, W_down1, W_red1, Wt2, Wa2, mb=256):
    m = N // 2
    return pl.pallas_call(
        _stage1_body,
        out_shape=(
            jax.ShapeDtypeStruct((B, m, 64), jnp.float32),
            jax.ShapeDtypeStruct((B, m, 256), jnp.float32),
            jax.ShapeDtypeStruct((B, m, 1), jnp.float32),
            jax.ShapeDtypeStruct((B, m, 512), jnp.bfloat16),
        ),
        grid=(B, m // mb),
        in_specs=[
            pl.BlockSpec((1, 8, N), lambda b, i: (b, 0, 0)),
            pl.BlockSpec((1, mb, 256), lambda b, i: (b, i, 0)),
            pl.BlockSpec((1, 1, N), lambda b, i: (b, 0, 0)),
            pl.BlockSpec((1, N, 256), lambda b, i: (b, 0, 0)),
            pl.BlockSpec((128, 128), lambda b, i: (0, 0)),
            pl.BlockSpec((128, 64), lambda b, i: (0, 0)),
            pl.BlockSpec((64, 256), lambda b, i: (0, 0)),
            pl.BlockSpec((256, 1), lambda b, i: (0, 0)),
        ],
        out_specs=(
            pl.BlockSpec((1, mb, 64), lambda b, i: (b, i, 0)),
            pl.BlockSpec((1, mb, 256), lambda b, i: (b, i, 0)),
            pl.BlockSpec((1, mb, 1), lambda b, i: (b, i, 0)),
            pl.BlockSpec((1, mb, 512), lambda b, i: (b, i, 0)),
        ),
    )(srcT, cent1, s0row, H0cat, W_down1, W_red1, Wt2, Wa2)


def _stage2(srcT2, cent2, s1row, H1cat, W_down2, W_red2, mb=512):
    m = N // 4
    ns = N // 2
    return pl.pallas_call(
        _stage2_body,
        out_shape=jax.ShapeDtypeStruct((B, m, 64), jnp.float32),
        grid=(B, m // mb),
        in_specs=[
            pl.BlockSpec((1, 8, ns), lambda b, i: (b, 0, 0)),
            pl.BlockSpec((1, mb, 384), lambda b, i: (b, i, 0)),
            pl.BlockSpec((1, 1, ns), lambda b, i: (b, 0, 0)),
            pl.BlockSpec((1, ns, 512), lambda b, i: (b, 0, 0)),
            pl.BlockSpec((256, 256), lambda b, i: (0, 0)),
            pl.BlockSpec((256, 64), lambda b, i: (0, 0)),
        ],
        out_specs=pl.BlockSpec((1, mb, 64), lambda b, i: (b, i, 0)),
    )(srcT2, cent2, s1row, H1cat, W_down2, W_red2)


def _gate_body(x1_ref, x2_ref, c1w1_ref, c1w2_ref, c2w1_ref, c2w2_ref,
               s1_ref, s2_ref):
    x1u = jnp.repeat(x1_ref[...], 2, axis=1)
    x2u = jnp.repeat(x2_ref[...], 4, axis=1)
    g1 = jnp.mean(x1u, axis=1)
    s1_ref[...] = jax.nn.sigmoid(
        _bdot(jax.nn.relu(_bdot(g1, c1w1_ref[...])), c1w2_ref[...]))
    g2 = jnp.mean(x2u, axis=1)
    s2_ref[...] = jax.nn.sigmoid(
        _bdot(jax.nn.relu(_bdot(g2, c2w1_ref[...])), c2w2_ref[...]))


def _gates(x1, x2, C1_W1, C1_W2, C2_W1, C2_W2):
    full = lambda *shape: pl.BlockSpec(shape, lambda: tuple(0 for _ in shape))
    return pl.pallas_call(
        _gate_body,
        out_shape=(jax.ShapeDtypeStruct((B, 64), jnp.float32),
                   jax.ShapeDtypeStruct((B, 64), jnp.float32)),
        grid=(),
        in_specs=[full(B, N // 2, 64), full(B, N // 4, 64),
                  full(64, 64), full(64, 64), full(64, 64), full(64, 64)],
        out_specs=(full(B, 64), full(B, 64)),
    )(x1, x2, C1_W1, C1_W2, C2_W1, C2_W2)


def _epi_y_body(x0_ref, x1_ref, x2_ref, s1_ref, s2_ref, wo1_ref,
                y_ref, ysum_ref, ysq_ref):
    b = pl.program_id(0)
    i = pl.program_id(1)
    x0 = x0_ref[0]
    x1u = jnp.repeat(x1_ref[0], 2, axis=0)
    x2u = jnp.repeat(x2_ref[0], 4, axis=0)
    sig1 = s1_ref[pl.ds(b, 1), :]
    sig2 = s2_ref[pl.ds(b, 1), :]
    x1e = x0 * sig1 + x1u
    x0e = x1e * sig2 + x2u
    fused = jnp.concatenate([x0e, x1u, x2u], axis=1)
    y = _bdot(fused, wo1_ref[...])
    y_ref[0] = y

    @pl.when(jnp.logical_and(b == 0, i == 0))
    def _():
        ysum_ref[...] = jnp.zeros_like(ysum_ref)
        ysq_ref[...] = jnp.zeros_like(ysq_ref)

    ysum_ref[...] += jnp.sum(y, axis=0, keepdims=True)
    ysq_ref[...] += jnp.sum(y * y, axis=0, keepdims=True)


def _epi_y(x0, x1, x2, sig1, sig2, W_out1, nb=2048):
    full = lambda *shape: pl.BlockSpec(shape, lambda b, i: tuple(0 for _ in shape))
    return pl.pallas_call(
        _epi_y_body,
        out_shape=(jax.ShapeDtypeStruct((B, N, 256), jnp.float32),
                   jax.ShapeDtypeStruct((1, 256), jnp.float32),
                   jax.ShapeDtypeStruct((1, 256), jnp.float32)),
        grid=(B, N // nb),
        in_specs=[
            pl.BlockSpec((1, nb, 64), lambda b, i: (b, i, 0)),
            pl.BlockSpec((1, nb // 2, 64), lambda b, i: (b, i, 0)),
            pl.BlockSpec((1, nb // 4, 64), lambda b, i: (b, i, 0)),
            full(B, 64), full(B, 64), full(192, 256),
        ],
        out_specs=(
            pl.BlockSpec((1, nb, 256), lambda b, i: (b, i, 0)),
            pl.BlockSpec((1, 256), lambda b, i: (0, 0)),
            pl.BlockSpec((1, 256), lambda b, i: (0, 0)),
        ),
    )(x0, x1, x2, sig1, sig2, W_out1)


def _epi_out_body(y_ref, ysum_ref, ysq_ref, gam_ref, bet_ref,
                  wo2_ref, bo2_ref, out_ref):
    cnt = jnp.float32(B * N)
    mu = ysum_ref[...] / cnt
    var = ysq_ref[...] / cnt - mu * mu
    y = y_ref[0]
    h = (y - mu) / jnp.sqrt(var + 1e-5) * gam_ref[...] + bet_ref[...]
    h = jax.nn.relu(h)
    out_ref[0] = _bdot(h, wo2_ref[...]) + bo2_ref[...]


def _epi_out(y, ysum, ysq, bn_gamma, bn_beta, W_out2, b_out2, nb=2048):
    nc = W_out2.shape[1]
    full = lambda *shape: pl.BlockSpec(shape, lambda b, i: tuple(0 for _ in shape))
    return pl.pallas_call(
        _epi_out_body,
        out_shape=jax.ShapeDtypeStruct((B, N, nc), jnp.float32),
        grid=(B, N // nb),
        in_specs=[
            pl.BlockSpec((1, nb, 256), lambda b, i: (b, i, 0)),
            full(1, 256), full(1, 256), full(1, 256), full(1, 256),
            full(256, nc), full(1, nc),
        ],
        out_specs=pl.BlockSpec((1, nb, nc), lambda b, i: (b, i, 0)),
    )(y, ysum, ysq, bn_gamma.reshape(1, 256), bn_beta.reshape(1, 256),
      W_out2, b_out2.reshape(1, nc))


def _epilogue(x0, x1, x2, C1_W1, C1_W2, C2_W1, C2_W2,
              W_out1, bn_gamma, bn_beta, W_out2, b_out2):
    sig1, sig2 = _gates(x1, x2, C1_W1, C1_W2, C2_W1, C2_W2)
    y, ysum, ysq = _epi_y(x0, x1, x2, sig1, sig2, W_out1)
    return _epi_out(y, ysum, ysq, bn_gamma, bn_beta, W_out2, b_out2)


def kernel(points, center_idx1, center_idx2, W_in, Wt1, Wa1, W_down1, W_red1,
           Wt2, Wa2, W_down2, W_red2, C1_W1, C1_W2, C2_W1, C2_W2,
           W_out1, bn_gamma, bn_beta, W_out2, b_out2):
    coords = points[:, :, :3]
    x0, H0, S0col, H0cat = _prologue(coords, W_in, Wt1, Wa1)

    srcT = jnp.pad(jnp.swapaxes(coords, 1, 2), ((0, 0), (0, 5), (0, 0)))
    s0row = jnp.swapaxes(S0col, 1, 2)

    cent1 = _gather_centers(coords, S0col, H0, center_idx1)
    x1, H1, S1col, H1cat = _stage1(srcT, cent1, s0row, H0cat,
                                   W_down1, W_red1, Wt2, Wa2)

    srcT2 = srcT[:, :, :N // 2]
    s1row = jnp.swapaxes(S1col, 1, 2)
    cent2 = _gather_centers(coords[:, :N // 2], S1col, H1, center_idx2)
    x2 = _stage2(srcT2, cent2, s1row, H1cat, W_down2, W_red2)

    return _epilogue(x0, x1, x2, C1_W1, C1_W2, C2_W1, C2_W2,
                     W_out1, bn_gamma, bn_beta, W_out2, b_out2)
